# baseline (speedup 1.0000x reference)
import functools

import jax
import jax.numpy as jnp
from jax import lax
from jax.experimental import pallas as pl
from jax.experimental.pallas import tpu as pltpu
from jax.experimental.pallas import tpu_sc as plsc

EPS = 1e-08

NC = 2
NS = 16
NW = NC * NS
L = 16


def _make_sc_kernel(B, A, N, CH):
    WPB = NW // B
    APW = A // WPB
    NCHUNK = APW // CH
    assert NCHUNK * CH == APW and N % (4 * L) == 0 and NCHUNK % 2 == 0
    CHN = CH * N

    mesh = plsc.VectorSubcoreMesh(
        core_axis_name="c", subcore_axis_name="s",
        num_cores=NC, num_subcores=NS)

    @functools.partial(
        pl.kernel,
        out_type=(
            jax.ShapeDtypeStruct((B * A * N,), jnp.float32),
            jax.ShapeDtypeStruct((B * A * N * 3,), jnp.float32),
        ),
        mesh=mesh,
        compiler_params=pltpu.CompilerParams(needs_layout_passes=False),
        scratch_types=[
            pltpu.VMEM((A * 3,), jnp.float32),
            pltpu.VMEM((CHN,), jnp.int32),
            pltpu.VMEM((CHN,), jnp.int32),
            pltpu.VMEM((CHN,), jnp.int32),
            pltpu.VMEM((CHN,), jnp.int32),
            pltpu.VMEM((CHN,), jnp.float32),
            pltpu.VMEM((CHN,), jnp.float32),
            pltpu.VMEM((CHN * 3,), jnp.float32),
            pltpu.VMEM((CHN * 3,), jnp.float32),
            pltpu.SemaphoreType.DMA,
            pltpu.SemaphoreType.DMA,
            pltpu.SemaphoreType.DMA,
            pltpu.SemaphoreType.DMA,
        ],
    )
    def sc_kernel(atm_hbm, nbr_hbm, msk_hbm, dist_hbm, dvec_hbm,
                  tbl, nb0, nb1, mk0, mk1, di0, di1, dv0, dv1,
                  isem0, isem1, osem0, osem1):
        nb = (nb0, nb1)
        mk = (mk0, mk1)
        di = (di0, di1)
        dv = (dv0, dv1)
        isem = (isem0, isem1)
        osem = (osem0, osem1)

        cid = lax.axis_index("c")
        sid = lax.axis_index("s")
        wid = sid * NC + cid
        b = wid // WPB
        a0 = (wid % WPB) * APW

        pltpu.sync_copy(atm_hbm.at[pl.ds(b * A * 3, A * 3)], tbl)

        base_slot = (b * A + a0) * N

        def start_in(ck, p):
            off = base_slot + ck * CHN
            pltpu.async_copy(nbr_hbm.at[pl.ds(off, CHN)], nb[p], isem[p])
            pltpu.async_copy(msk_hbm.at[pl.ds(off, CHN)], mk[p], isem[p])

        def wait_in(p):
            pltpu.make_async_copy(nbr_hbm.at[pl.ds(0, CHN)], nb[p], isem[p]).wait()
            pltpu.make_async_copy(msk_hbm.at[pl.ds(0, CHN)], mk[p], isem[p]).wait()

        def start_out(ck, p):
            off = base_slot + ck * CHN
            pltpu.async_copy(di[p], dist_hbm.at[pl.ds(off, CHN)], osem[p])
            pltpu.async_copy(dv[p], dvec_hbm.at[pl.ds(off * 3, CHN * 3)], osem[p])

        def wait_out(p):
            pltpu.make_async_copy(di[p], dist_hbm.at[pl.ds(0, CHN)], osem[p]).wait()
            pltpu.make_async_copy(dv[p], dvec_hbm.at[pl.ds(0, CHN * 3)], osem[p]).wait()

        iota = lax.iota(jnp.int32, L)
        i4 = iota * 4
        i12 = iota * 12

        def compute(ck, p):
            nb_v, mk_v, di_v, dv_v = nb[p], mk[p], di[p], dv[p]
            ac = a0 + ck * CH

            def center_body(lc, carry2):
                o0 = lc * N
                C4 = range(4)
                av = jnp.full((L,), (ac + lc) * 3, dtype=jnp.int32)
                cx = plsc.load_gather(tbl, [av])
                cy = plsc.load_gather(tbl, [av + 1])
                cz = plsc.load_gather(tbl, [av + 2])
                sl = [i4 + (o0 + c) for c in C4]
                mv = [plsc.load_gather(mk_v, [sl[c]]) for c in C4]
                idxv = [plsc.load_gather(nb_v, [sl[c]]) for c in C4]
                ix = [idxv[c] * 3 for c in C4]
                gx = [plsc.load_gather(tbl, [ix[c]]) for c in C4]
                gy = [plsc.load_gather(tbl, [ix[c] + 1]) for c in C4]
                gz = [plsc.load_gather(tbl, [ix[c] + 2]) for c in C4]
                dx = [gx[c] - cx for c in C4]
                dy = [gy[c] - cy for c in C4]
                dz = [gz[c] - cz for c in C4]
                ss = [dx[c] * dx[c] + dy[c] * dy[c] + dz[c] * dz[c]
                      for c in C4]
                seed = [jnp.int32(0x5F3759DF) - lax.shift_right_logical(
                    plsc.bitcast(ss[c], jnp.int32), 1) for c in C4]
                y = [plsc.bitcast(seed[c], jnp.float32) for c in C4]
                h = [ss[c] * 0.5 for c in C4]
                y = [y[c] * (1.5 - h[c] * y[c] * y[c]) for c in C4]
                y = [y[c] * (1.5 - h[c] * y[c] * y[c]) for c in C4]
                d = [ss[c] * y[c] for c in C4]
                d = [jnp.where(ss[c] > 0.0, d[c], 0.0) for c in C4]
                m = [mv[c] != 0 for c in C4]
                for c in C4:
                    plsc.store_scatter(di_v, [sl[c]],
                                       jnp.where(m[c], d[c], 0.0))
                r = [1.0 / jnp.where(m[c], d[c] + EPS, 1.0) for c in C4]
                sidx = [i12 + (3 * (o0 + c)) for c in C4]
                for c in C4:
                    plsc.store_scatter(dv_v, [sidx[c]], dx[c] * r[c])
                    plsc.store_scatter(dv_v, [sidx[c] + 1], dy[c] * r[c])
                    plsc.store_scatter(dv_v, [sidx[c] + 2], dz[c] * r[c])
                return carry2

            lax.fori_loop(0, CH, center_body, 0)

        start_in(0, 0)
        start_in(1, 1)
        for ck in range(NCHUNK):
            p = ck % 2
            wait_in(p)
            if ck >= 2:
                wait_out(p)
            compute(ck, p)
            start_out(ck, p)
            if ck + 2 < NCHUNK:
                start_in(ck + 2, p)
        wait_out(0)
        wait_out(1)

    return sc_kernel


def kernel(atoms, neighbors, neighbor_mask):
    B, A, _ = atoms.shape
    N = neighbors.shape[-1]
    atoms_flat = atoms.reshape(B * A * 3)
    nbr_flat = neighbors.reshape(B * A * N)
    msk_flat = neighbor_mask.astype(jnp.int32).reshape(B * A * N)

    sc_kernel = _make_sc_kernel(B, A, N, CH=128)
    dist_flat, dvec_flat = sc_kernel(atoms_flat, nbr_flat, msk_flat)

    distances = dist_flat.reshape(B, A, N)
    distance_vector = dvec_flat.reshape(B, A, N, 3)
    return (distances, distance_vector, neighbors, neighbor_mask)
